# Initial kernel scaffold; baseline (speedup 1.0000x reference)
import jax
import jax.numpy as jnp
from jax.experimental import pallas as pl


def kernel(x_list, edge_index, Wx, Wh, b, W_im, b_im):
    raise NotImplementedError("write your pallas kernel here")


# speedup vs baseline: 11.4374x; 11.4374x over previous
import functools

import jax
import jax.numpy as jnp
from jax import lax
from jax.experimental import pallas as pl
from jax.experimental.pallas import tpu as pltpu
from jax.experimental.pallas import tpu_sc as plsc

N = 10000
T = 3
E = 320000
D = 128
DH = 64
NC = 2
NS = 16
EC = 128
NBLK = E // EC
BLK_PER_TILE = -(-NBLK // NS)
ROWS_PER_TILE = 640
RC = 80

_mesh = plsc.VectorSubcoreMesh(core_axis_name="c", subcore_axis_name="s",
                               num_cores=NC, num_subcores=NS)
_sc_params = pltpu.CompilerParams(use_tc_tiling_on_sc=False,
                                  needs_layout_passes=False)


def _agg_body(compute_deg, h2, src, dst, out, *rest):
    if compute_deg:
        (recip, srcbuf, dstbuf, idxbuf, rows, zbuf, obuf, onesbuf, dbuf,
         rbuf, acc_sp, deg_sp, sem) = rest
    else:
        (srcbuf, dstbuf, idxbuf, rows, zbuf, obuf, acc_sp, sem) = rest
    fh = lax.axis_index("c")
    s = lax.axis_index("s")

    zero16 = jnp.zeros((16,), jnp.float32)
    for k in range(4):
        zbuf[0, pl.ds(16 * k, 16)] = zero16

    def _fill(i, _):
        for k in range(4):
            zbuf[i, pl.ds(16 * k, 16)] = zero16
        return 0
    lax.fori_loop(1, 125, _fill, 0)

    def _zero_chunk(m, _):
        r0 = s * 625 + 125 * m
        pltpu.sync_copy(zbuf, acc_sp.at[pl.ds(r0, 125), :])
        return 0
    lax.fori_loop(0, 5, _zero_chunk, 0)

    if compute_deg:
        for k in range(5):
            rbuf[pl.ds(16 * k, 16)] = zero16

        def _zero_deg(j, _):
            k = s + 16 * j

            @pl.when(k < 125)
            def _():
                pltpu.sync_copy(rbuf, deg_sp.at[pl.ds(RC * k, RC)])
            return 0
        lax.fori_loop(0, 8, _zero_deg, 0)
        ones16 = jnp.ones((16,), jnp.float32)
        for k in range(8):
            onesbuf[pl.ds(16 * k, 16)] = ones16

    plsc.subcore_barrier()

    def _edge_chunk(j, _):
        b = s + NS * j

        @pl.when(b < NBLK)
        def _():
            pltpu.sync_copy(src.at[pl.ds(EC * b, EC)], srcbuf)
            pltpu.sync_copy(dst.at[pl.ds(EC * b, EC)], dstbuf)
            for k in range(EC // 16):
                idxbuf[pl.ds(16 * k, 16)] = (
                    srcbuf[pl.ds(16 * k, 16)] * 2 + fh)
            pltpu.async_copy(h2.at[idxbuf], rows, sem).wait()
            pltpu.sync_copy(rows, acc_sp.at[dstbuf], add=True)
            if compute_deg:
                pltpu.sync_copy(onesbuf, deg_sp.at[dstbuf], add=True)
        return 0
    lax.fori_loop(0, BLK_PER_TILE, _edge_chunk, 0)

    plsc.subcore_barrier()

    def _write_chunk(m, _):
        r0 = s * ROWS_PER_TILE + RC * m

        @pl.when(r0 < N)
        def _():
            pltpu.sync_copy(acc_sp.at[pl.ds(r0, RC), :], obuf)
            pltpu.sync_copy(obuf, out.at[pl.ds(r0, RC), pl.ds(DH * fh, DH)])
        return 0
    lax.fori_loop(0, 8, _write_chunk, 0)

    if compute_deg:
        def _recip_chunk(j, _):
            k = (s * NC + fh) + NS * NC * j

            @pl.when(k < N // RC)
            def _():
                pltpu.sync_copy(deg_sp.at[pl.ds(RC * k, RC)], dbuf)
                for i in range(RC // 16):
                    d = dbuf[pl.ds(16 * i, 16)]
                    rbuf[pl.ds(16 * i, 16)] = 1.0 / jnp.maximum(d, 1.0)
                pltpu.sync_copy(rbuf, recip.at[pl.ds(RC * k, RC)])
            return 0
        lax.fori_loop(0, 4, _recip_chunk, 0)


_agg_scratch_common = dict(
    srcbuf=pltpu.VMEM((EC,), jnp.int32),
    dstbuf=pltpu.VMEM((EC,), jnp.int32),
    idxbuf=pltpu.VMEM((EC,), jnp.int32),
    rows=pltpu.VMEM((EC, DH), jnp.float32),
    zbuf=pltpu.VMEM((125, DH), jnp.float32),
    obuf=pltpu.VMEM((RC, DH), jnp.float32),
)

_agg_deg = pl.kernel(
    functools.partial(_agg_body, True),
    out_type=(jax.ShapeDtypeStruct((N, D), jnp.float32),
              jax.ShapeDtypeStruct((N,), jnp.float32)),
    mesh=_mesh,
    scratch_types=[
        _agg_scratch_common["srcbuf"], _agg_scratch_common["dstbuf"],
        _agg_scratch_common["idxbuf"], _agg_scratch_common["rows"],
        _agg_scratch_common["zbuf"], _agg_scratch_common["obuf"],
        pltpu.VMEM((EC,), jnp.float32),
        pltpu.VMEM((RC,), jnp.float32),
        pltpu.VMEM((RC,), jnp.float32),
        pltpu.VMEM_SHARED((N, DH), jnp.float32),
        pltpu.VMEM_SHARED((N,), jnp.float32),
        pltpu.SemaphoreType.DMA,
    ],
    compiler_params=_sc_params,
)

_agg_plain = pl.kernel(
    functools.partial(_agg_body, False),
    out_type=jax.ShapeDtypeStruct((N, D), jnp.float32),
    mesh=_mesh,
    scratch_types=[
        _agg_scratch_common["srcbuf"], _agg_scratch_common["dstbuf"],
        _agg_scratch_common["idxbuf"], _agg_scratch_common["rows"],
        _agg_scratch_common["zbuf"], _agg_scratch_common["obuf"],
        pltpu.VMEM_SHARED((N, DH), jnp.float32),
        pltpu.SemaphoreType.DMA,
    ],
    compiler_params=_sc_params,
)


def _shuffle_body(hxT, perms, shT, hbuf, pbuf, obuf, sem):
    wid = lax.axis_index("s") * NC + lax.axis_index("c")
    for jj in range(4):
        j = wid * 4 + jj
        pltpu.sync_copy(hxT.at[j], hbuf)
        pltpu.sync_copy(perms.at[j], pbuf)

        def _blk(k, _):
            idxv = pbuf[pl.ds(16 * k, 16)]
            obuf[pl.ds(16 * k, 16)] = plsc.load_gather(hbuf, [idxv])
            return 0
        lax.fori_loop(0, N // 16, _blk, 0)
        pltpu.sync_copy(obuf, shT.at[j])


_shuffle = pl.kernel(
    _shuffle_body,
    out_type=jax.ShapeDtypeStruct((D, N), jnp.float32),
    mesh=_mesh,
    scratch_types=[
        pltpu.VMEM((N,), jnp.float32),
        pltpu.VMEM((N,), jnp.int32),
        pltpu.VMEM((N,), jnp.float32),
        pltpu.SemaphoreType.DMA,
    ],
    compiler_params=_sc_params,
)


BN = 1000
GRID = N // BN


def _scale2_body(x1, x2, rec, y1, y2):
    r = rec[...]
    y1[...] = x1[...] * r
    y2[...] = x2[...] * r


_scale2 = pl.pallas_call(
    _scale2_body,
    grid=(GRID,),
    in_specs=[pl.BlockSpec((BN, D), lambda i: (i, 0)),
              pl.BlockSpec((BN, D), lambda i: (i, 0)),
              pl.BlockSpec((BN, 1), lambda i: (i, 0))],
    out_specs=[pl.BlockSpec((BN, D), lambda i: (i, 0)),
               pl.BlockSpec((BN, D), lambda i: (i, 0))],
    out_shape=[jax.ShapeDtypeStruct((N, D), jnp.float32),
               jax.ShapeDtypeStruct((N, D), jnp.float32)],
)


def _gru_body(ax2r, ah2r, rec, hx, Wx, Wh, bb, Wim,
              hxn, Wc, csum):
    r_deg = rec[...]
    ax = ax2r[...] * r_deg
    ah = ah2r[...] * r_deg
    wx = Wx[...]
    wh = Wh[...]
    b = bb[...]
    zr = (jnp.dot(ax, wx[:, :2 * D], preferred_element_type=jnp.float32)
          + jnp.dot(ah, wh[:, :2 * D], preferred_element_type=jnp.float32)
          + b[:, :2 * D])
    z = jax.nn.sigmoid(zr[:, :D])
    r = jax.nn.sigmoid(zr[:, D:])
    ht = jnp.tanh(
        jnp.dot(ax, wx[:, 2 * D:], preferred_element_type=jnp.float32)
        + jnp.dot(r * ah, wh[:, 2 * D:], preferred_element_type=jnp.float32)
        + b[:, 2 * D:])
    hn = (1.0 - z) * hx[...] + z * ht
    hxn[...] = hn
    i = pl.program_id(0)

    @pl.when(i == 0)
    def _():
        csum[...] = jnp.sum(hn, axis=0, keepdims=True)

    @pl.when(i > 0)
    def _():
        csum[...] += jnp.sum(hn, axis=0, keepdims=True)

    @pl.when(i == GRID - 1)
    def _():
        c = jax.nn.sigmoid(csum[...] / N)
        Wc[...] = jnp.dot(c, Wim[...].T, preferred_element_type=jnp.float32)


_gru = pl.pallas_call(
    _gru_body,
    grid=(GRID,),
    in_specs=[pl.BlockSpec((BN, D), lambda i: (i, 0)),
              pl.BlockSpec((BN, D), lambda i: (i, 0)),
              pl.BlockSpec((BN, 1), lambda i: (i, 0)),
              pl.BlockSpec((BN, D), lambda i: (i, 0)),
              pl.BlockSpec((D, 3 * D), lambda i: (0, 0)),
              pl.BlockSpec((D, 3 * D), lambda i: (0, 0)),
              pl.BlockSpec((1, 3 * D), lambda i: (0, 0)),
              pl.BlockSpec((D, D), lambda i: (0, 0))],
    out_specs=[pl.BlockSpec((BN, D), lambda i: (i, 0)),
               pl.BlockSpec((1, D), lambda i: (0, 0))],
    out_shape=[jax.ShapeDtypeStruct((N, D), jnp.float32),
               jax.ShapeDtypeStruct((1, D), jnp.float32)],
    scratch_shapes=[pltpu.VMEM((1, D), jnp.float32)],
)


def _tp_body(x, y):
    y[...] = x[...].T


_tp = pl.pallas_call(
    _tp_body,
    out_shape=jax.ShapeDtypeStruct((D, N), jnp.float32),
)


def _fin_body(hxn, shT, Wc, bim, pos, neg):
    wc = Wc[...]
    bv = bim[0, 0]
    pos[...] = jax.nn.sigmoid(
        jnp.sum(hxn[...] * wc, axis=1, keepdims=True) + bv)
    neg[...] = jax.nn.sigmoid(
        jnp.sum(shT[...] * wc.reshape(D, 1), axis=0, keepdims=True) + bv)


_fin = pl.pallas_call(
    _fin_body,
    out_shape=[jax.ShapeDtypeStruct((N, 1), jnp.float32),
               jax.ShapeDtypeStruct((1, N), jnp.float32)],
)


def kernel(x_list, edge_index, Wx, Wh, b, W_im, b_im):
    x_list = x_list.astype(jnp.float32)
    Wx = Wx.astype(jnp.float32)
    Wh = Wh.astype(jnp.float32)
    bb = b.astype(jnp.float32).reshape(1, 3 * D)
    base = jax.random.key(42)

    hx = jnp.zeros((N, D), jnp.float32)
    zeros_nd = jnp.zeros((N, D), jnp.float32)
    hx_out, pos_out, neg_out = [], [], []
    for t in range(T):
        src = edge_index[t, 0].astype(jnp.int32)
        dst = edge_index[t, 1].astype(jnp.int32)
        x2 = x_list[t].reshape(2 * N, DH)
        s1x, recip = _agg_deg(x2, src, dst)
        if t == 0:
            s1h = zeros_nd
        else:
            s1h = _agg_plain(hx.reshape(2 * N, DH), src, dst)
        rec2d = recip.reshape(N, 1)
        x1n, h1n = _scale2(s1x, s1h, rec2d)
        ax2r = _agg_plain(x1n.reshape(2 * N, DH), src, dst)
        if t == 0:
            ah2r = zeros_nd
        else:
            ah2r = _agg_plain(h1n.reshape(2 * N, DH), src, dst)
        hx, Wc = _gru(ax2r, ah2r, rec2d, hx, Wx, Wh, bb, W_im[t])
        hxT = _tp(hx)

        keys = jax.random.split(jax.random.fold_in(base, t), D)
        perms = jax.vmap(lambda k: jax.random.permutation(k, N))(keys)
        perms = perms.astype(jnp.int32)
        shT = _shuffle(hxT, perms)
        pos2d, neg2d = _fin(hx, shT, Wc, b_im[t].reshape(1, 1))
        hx_out.append(hx)
        pos_out.append(pos2d.reshape(N))
        neg_out.append(neg2d.reshape(N))
    return jnp.stack(hx_out), jnp.stack(pos_out), jnp.stack(neg_out)


# speedup vs baseline: 40.5559x; 3.5459x over previous
import functools

import jax
import jax.numpy as jnp
import numpy as np
from jax import lax
from jax.experimental import pallas as pl
from jax.experimental.pallas import tpu as pltpu
from jax.experimental.pallas import tpu_sc as plsc

N = 10000
T = 3
E = 320000
D = 128
DH = 64
NC = 2
NS = 16
EC = 128
NBLK = E // EC
KB = 4
NSS = NBLK // KB
SS_PER_TILE = -(-NSS // NS)
ROWS_PER_TILE = 640
RC = 80

_mesh = plsc.VectorSubcoreMesh(core_axis_name="c", subcore_axis_name="s",
                               num_cores=NC, num_subcores=NS)
_sc_params = pltpu.CompilerParams(use_tc_tiling_on_sc=False,
                                  needs_layout_passes=False)


def _agg_body(compute_deg, h2, src3, dst3, out, *rest):
    if compute_deg:
        (recip, srcbuf, dstbuf, idxbuf, rows, zbuf, obuf, onesbuf, dbuf,
         rbuf, acc_sp, deg_sp, gsem, ssem) = rest
    else:
        (srcbuf, dstbuf, idxbuf, rows, zbuf, obuf, acc_sp, gsem, ssem) = rest
    fh = lax.axis_index("c")
    s = lax.axis_index("s")

    zero16 = jnp.zeros((16,), jnp.float32)
    for k in range(4):
        zbuf[0, pl.ds(16 * k, 16)] = zero16

    def _fill(i, _):
        for k in range(4):
            zbuf[i, pl.ds(16 * k, 16)] = zero16
        return 0
    lax.fori_loop(1, 125, _fill, 0)

    def _zero_chunk(m, _):
        r0 = s * 625 + 125 * m
        pltpu.sync_copy(zbuf, acc_sp.at[pl.ds(r0, 125), :])
        return 0
    lax.fori_loop(0, 5, _zero_chunk, 0)

    if compute_deg:
        for k in range(5):
            rbuf[pl.ds(16 * k, 16)] = zero16

        def _zero_deg(j, _):
            k = s + 16 * j

            @pl.when(k < 125)
            def _():
                pltpu.sync_copy(rbuf, deg_sp.at[pl.ds(RC * k, RC)])
            return 0
        lax.fori_loop(0, 8, _zero_deg, 0)
        ones16 = jnp.ones((16,), jnp.float32)
        for k in range(8):
            onesbuf[pl.ds(16 * k, 16)] = ones16

    plsc.subcore_barrier()

    def _superstep(gi, _):
        g = s + NS * gi

        @pl.when(g < NSS)
        def _():
            pltpu.sync_copy(src3.at[pl.ds(KB * g, KB), :], srcbuf)
            pltpu.sync_copy(dst3.at[pl.ds(KB * g, KB), :], dstbuf)
            for j in range(KB):
                for k in range(EC // 16):
                    idxbuf[j, pl.ds(16 * k, 16)] = (
                        srcbuf[j, pl.ds(16 * k, 16)] * 2 + fh)
            gds = [pltpu.async_copy(h2.at[idxbuf.at[j]], rows.at[j], gsem)
                   for j in range(KB)]
            sds = []
            for j in range(KB):
                gds[j].wait()
                sds.append(pltpu.async_copy(
                    rows.at[j], acc_sp.at[dstbuf.at[j]], ssem, add=True))
                if compute_deg:
                    sds.append(pltpu.async_copy(
                        onesbuf, deg_sp.at[dstbuf.at[j]], ssem, add=True))
            for d in sds:
                d.wait()
        return 0
    lax.fori_loop(0, SS_PER_TILE, _superstep, 0)

    plsc.subcore_barrier()

    def _write_chunk(m, _):
        r0 = s * ROWS_PER_TILE + RC * m

        @pl.when(r0 < N)
        def _():
            pltpu.sync_copy(acc_sp.at[pl.ds(r0, RC), :], obuf)
            pltpu.sync_copy(obuf, out.at[pl.ds(r0, RC), pl.ds(DH * fh, DH)])
        return 0
    lax.fori_loop(0, 8, _write_chunk, 0)

    if compute_deg:
        def _recip_chunk(j, _):
            k = (s * NC + fh) + NS * NC * j

            @pl.when(k < N // RC)
            def _():
                pltpu.sync_copy(deg_sp.at[pl.ds(RC * k, RC)], dbuf)
                for i in range(RC // 16):
                    d = dbuf[pl.ds(16 * i, 16)]
                    rbuf[pl.ds(16 * i, 16)] = 1.0 / jnp.maximum(d, 1.0)
                pltpu.sync_copy(rbuf, recip.at[pl.ds(RC * k, RC)])
            return 0
        lax.fori_loop(0, 4, _recip_chunk, 0)


_agg_scratch_common = dict(
    srcbuf=pltpu.VMEM((KB, EC), jnp.int32),
    dstbuf=pltpu.VMEM((KB, EC), jnp.int32),
    idxbuf=pltpu.VMEM((KB, EC), jnp.int32),
    rows=pltpu.VMEM((KB, EC, DH), jnp.float32),
    zbuf=pltpu.VMEM((125, DH), jnp.float32),
    obuf=pltpu.VMEM((RC, DH), jnp.float32),
)

_agg_deg = pl.kernel(
    functools.partial(_agg_body, True),
    out_type=(jax.ShapeDtypeStruct((N, D), jnp.float32),
              jax.ShapeDtypeStruct((N,), jnp.float32)),
    mesh=_mesh,
    scratch_types=[
        _agg_scratch_common["srcbuf"], _agg_scratch_common["dstbuf"],
        _agg_scratch_common["idxbuf"], _agg_scratch_common["rows"],
        _agg_scratch_common["zbuf"], _agg_scratch_common["obuf"],
        pltpu.VMEM((EC,), jnp.float32),
        pltpu.VMEM((RC,), jnp.float32),
        pltpu.VMEM((RC,), jnp.float32),
        pltpu.VMEM_SHARED((N, DH), jnp.float32),
        pltpu.VMEM_SHARED((N,), jnp.float32),
        pltpu.SemaphoreType.DMA,
        pltpu.SemaphoreType.DMA,
    ],
    compiler_params=_sc_params,
)

_agg_plain = pl.kernel(
    functools.partial(_agg_body, False),
    out_type=jax.ShapeDtypeStruct((N, D), jnp.float32),
    mesh=_mesh,
    scratch_types=[
        _agg_scratch_common["srcbuf"], _agg_scratch_common["dstbuf"],
        _agg_scratch_common["idxbuf"], _agg_scratch_common["rows"],
        _agg_scratch_common["zbuf"], _agg_scratch_common["obuf"],
        pltpu.VMEM_SHARED((N, DH), jnp.float32),
        pltpu.SemaphoreType.DMA,
        pltpu.SemaphoreType.DMA,
    ],
    compiler_params=_sc_params,
)


def _shuffle_body(hxT, perms, shT, hbuf, pbuf, obuf, sem):
    wid = lax.axis_index("s") * NC + lax.axis_index("c")
    for jj in range(4):
        j = wid * 4 + jj
        pltpu.sync_copy(hxT.at[j], hbuf)
        pltpu.sync_copy(perms.at[j], pbuf)

        def _blk(k, _):
            idxv = pbuf[pl.ds(16 * k, 16)]
            obuf[pl.ds(16 * k, 16)] = plsc.load_gather(hbuf, [idxv])
            return 0
        lax.fori_loop(0, N // 16, _blk, 0)
        pltpu.sync_copy(obuf, shT.at[j])


_shuffle = pl.kernel(
    _shuffle_body,
    out_type=jax.ShapeDtypeStruct((D, N), jnp.float32),
    mesh=_mesh,
    scratch_types=[
        pltpu.VMEM((N,), jnp.float32),
        pltpu.VMEM((N,), jnp.int32),
        pltpu.VMEM((N,), jnp.float32),
        pltpu.SemaphoreType.DMA,
    ],
    compiler_params=_sc_params,
)


BN = 1000
GRID = N // BN


def _scale2_body(x1, x2, rec, y1, y2):
    r = rec[...]
    y1[...] = x1[...] * r
    y2[...] = x2[...] * r


_scale2 = pl.pallas_call(
    _scale2_body,
    grid=(GRID,),
    in_specs=[pl.BlockSpec((BN, D), lambda i: (i, 0)),
              pl.BlockSpec((BN, D), lambda i: (i, 0)),
              pl.BlockSpec((BN, 1), lambda i: (i, 0))],
    out_specs=[pl.BlockSpec((BN, D), lambda i: (i, 0)),
               pl.BlockSpec((BN, D), lambda i: (i, 0))],
    out_shape=[jax.ShapeDtypeStruct((N, D), jnp.float32),
               jax.ShapeDtypeStruct((N, D), jnp.float32)],
)


def _gru_body(ax2r, ah2r, rec, hx, Wx, Wh, bb, Wim,
              hxn, Wc, csum):
    r_deg = rec[...]
    ax = ax2r[...] * r_deg
    ah = ah2r[...] * r_deg
    wx = Wx[...]
    wh = Wh[...]
    b = bb[...]
    zr = (jnp.dot(ax, wx[:, :2 * D], preferred_element_type=jnp.float32)
          + jnp.dot(ah, wh[:, :2 * D], preferred_element_type=jnp.float32)
          + b[:, :2 * D])
    z = jax.nn.sigmoid(zr[:, :D])
    r = jax.nn.sigmoid(zr[:, D:])
    ht = jnp.tanh(
        jnp.dot(ax, wx[:, 2 * D:], preferred_element_type=jnp.float32)
        + jnp.dot(r * ah, wh[:, 2 * D:], preferred_element_type=jnp.float32)
        + b[:, 2 * D:])
    hn = (1.0 - z) * hx[...] + z * ht
    hxn[...] = hn
    i = pl.program_id(0)

    @pl.when(i == 0)
    def _():
        csum[...] = jnp.sum(hn, axis=0, keepdims=True)

    @pl.when(i > 0)
    def _():
        csum[...] += jnp.sum(hn, axis=0, keepdims=True)

    @pl.when(i == GRID - 1)
    def _():
        c = jax.nn.sigmoid(csum[...] / N)
        Wc[...] = jnp.dot(c, Wim[...].T, preferred_element_type=jnp.float32)


_gru = pl.pallas_call(
    _gru_body,
    grid=(GRID,),
    in_specs=[pl.BlockSpec((BN, D), lambda i: (i, 0)),
              pl.BlockSpec((BN, D), lambda i: (i, 0)),
              pl.BlockSpec((BN, 1), lambda i: (i, 0)),
              pl.BlockSpec((BN, D), lambda i: (i, 0)),
              pl.BlockSpec((D, 3 * D), lambda i: (0, 0)),
              pl.BlockSpec((D, 3 * D), lambda i: (0, 0)),
              pl.BlockSpec((1, 3 * D), lambda i: (0, 0)),
              pl.BlockSpec((D, D), lambda i: (0, 0))],
    out_specs=[pl.BlockSpec((BN, D), lambda i: (i, 0)),
               pl.BlockSpec((1, D), lambda i: (0, 0))],
    out_shape=[jax.ShapeDtypeStruct((N, D), jnp.float32),
               jax.ShapeDtypeStruct((1, D), jnp.float32)],
    scratch_shapes=[pltpu.VMEM((1, D), jnp.float32)],
)


def _tp_body(x, y):
    y[...] = x[...].T


_tp = pl.pallas_call(
    _tp_body,
    out_shape=jax.ShapeDtypeStruct((D, N), jnp.float32),
)


def _fin_body(hxn, shT, Wc, bim, pos, neg):
    wc = Wc[...]
    bv = bim[0, 0]
    pos[...] = jax.nn.sigmoid(
        jnp.sum(hxn[...] * wc, axis=1, keepdims=True) + bv)
    neg[...] = jax.nn.sigmoid(
        jnp.sum(shT[...] * wc.reshape(D, 1), axis=0, keepdims=True) + bv)


_fin = pl.pallas_call(
    _fin_body,
    out_shape=[jax.ShapeDtypeStruct((N, 1), jnp.float32),
               jax.ShapeDtypeStruct((1, N), jnp.float32)],
)


@functools.cache
def _dgi_perms():
    with jax.ensure_compile_time_eval():
        with jax.default_device(jax.devices("cpu")[0]):
            base = jax.random.key(42)
            out = []
            for t in range(T):
                keys = jax.random.split(jax.random.fold_in(base, t), D)
                perms = jax.vmap(
                    lambda k: jax.random.permutation(k, N))(keys)
                out.append(np.asarray(perms).astype(np.int32))
    return out


def kernel(x_list, edge_index, Wx, Wh, b, W_im, b_im):
    x_list = x_list.astype(jnp.float32)
    Wx = Wx.astype(jnp.float32)
    Wh = Wh.astype(jnp.float32)
    bb = b.astype(jnp.float32).reshape(1, 3 * D)

    hx = jnp.zeros((N, D), jnp.float32)
    zeros_nd = jnp.zeros((N, D), jnp.float32)
    hx_out, pos_out, neg_out = [], [], []
    for t in range(T):
        src = edge_index[t, 0].astype(jnp.int32).reshape(NBLK, EC)
        dst = edge_index[t, 1].astype(jnp.int32).reshape(NBLK, EC)
        x2 = x_list[t].reshape(2 * N, DH)
        s1x, recip = _agg_deg(x2, src, dst)
        if t == 0:
            s1h = zeros_nd
        else:
            s1h = _agg_plain(hx.reshape(2 * N, DH), src, dst)
        rec2d = recip.reshape(N, 1)
        x1n, h1n = _scale2(s1x, s1h, rec2d)
        ax2r = _agg_plain(x1n.reshape(2 * N, DH), src, dst)
        if t == 0:
            ah2r = zeros_nd
        else:
            ah2r = _agg_plain(h1n.reshape(2 * N, DH), src, dst)
        hx, Wc = _gru(ax2r, ah2r, rec2d, hx, Wx, Wh, bb, W_im[t])
        hxT = _tp(hx)

        shT = _shuffle(hxT, _dgi_perms()[t])
        pos2d, neg2d = _fin(hx, shT, Wc, b_im[t].reshape(1, 1))
        hx_out.append(hx)
        pos_out.append(pos2d.reshape(N))
        neg_out.append(neg2d.reshape(N))
    return jnp.stack(hx_out), jnp.stack(pos_out), jnp.stack(neg_out)


# speedup vs baseline: 43.1314x; 1.0635x over previous
import functools

import jax
import jax.numpy as jnp
import numpy as np
from jax import lax
from jax.experimental import pallas as pl
from jax.experimental.pallas import tpu as pltpu
from jax.experimental.pallas import tpu_sc as plsc

N = 10000
T = 3
E = 320000
D = 128
DH = 64
NC = 2
NS = 16
EC = 128
NBLK = E // EC
KB = 5
NSS = NBLK // KB
SS_PER_TILE = -(-NSS // NS)
ROWS_PER_TILE = 640
RC = 80

_mesh = plsc.VectorSubcoreMesh(core_axis_name="c", subcore_axis_name="s",
                               num_cores=NC, num_subcores=NS)
_sc_params = pltpu.CompilerParams(use_tc_tiling_on_sc=False,
                                  needs_layout_passes=False)


def _agg_body(compute_deg, h2, src3, dst3, out, *rest):
    if compute_deg:
        (recip, srcbuf, dstbuf, idxbuf, rows, zbuf, obuf, onesbuf, dbuf,
         rbuf, acc_sp, deg_sp, gsem, ssem) = rest
    else:
        (srcbuf, dstbuf, idxbuf, rows, zbuf, obuf, acc_sp, gsem, ssem) = rest
    fh = lax.axis_index("c")
    s = lax.axis_index("s")

    zero16 = jnp.zeros((16,), jnp.float32)
    for k in range(4):
        zbuf[0, pl.ds(16 * k, 16)] = zero16

    def _fill(i, _):
        for k in range(4):
            zbuf[i, pl.ds(16 * k, 16)] = zero16
        return 0
    lax.fori_loop(1, 125, _fill, 0)

    def _zero_chunk(m, _):
        r0 = s * 625 + 125 * m
        pltpu.sync_copy(zbuf, acc_sp.at[pl.ds(r0, 125), :])
        return 0
    lax.fori_loop(0, 5, _zero_chunk, 0)

    if compute_deg:
        for k in range(5):
            rbuf[pl.ds(16 * k, 16)] = zero16

        def _zero_deg(j, _):
            k = s + 16 * j

            @pl.when(k < 125)
            def _():
                pltpu.sync_copy(rbuf, deg_sp.at[pl.ds(RC * k, RC)])
            return 0
        lax.fori_loop(0, 8, _zero_deg, 0)
        ones16 = jnp.ones((16,), jnp.float32)
        for k in range(8):
            onesbuf[pl.ds(16 * k, 16)] = ones16

    plsc.subcore_barrier()

    def _superstep(gi, _):
        g = s + NS * gi

        @pl.when(g < NSS)
        def _():
            pltpu.sync_copy(src3.at[pl.ds(KB * g, KB), :], srcbuf)
            pltpu.sync_copy(dst3.at[pl.ds(KB * g, KB), :], dstbuf)
            for j in range(KB):
                for k in range(EC // 16):
                    idxbuf[j, pl.ds(16 * k, 16)] = (
                        srcbuf[j, pl.ds(16 * k, 16)] * 2 + fh)
            gds = [pltpu.async_copy(h2.at[idxbuf.at[j]], rows.at[j], gsem)
                   for j in range(KB)]
            sds = []
            for j in range(KB):
                gds[j].wait()
                sds.append(pltpu.async_copy(
                    rows.at[j], acc_sp.at[dstbuf.at[j]], ssem, add=True))
                if compute_deg:
                    sds.append(pltpu.async_copy(
                        onesbuf, deg_sp.at[dstbuf.at[j]], ssem, add=True))
            for d in sds:
                d.wait()
        return 0
    lax.fori_loop(0, SS_PER_TILE, _superstep, 0)

    plsc.subcore_barrier()

    def _write_chunk(m, _):
        r0 = s * ROWS_PER_TILE + RC * m

        @pl.when(r0 < N)
        def _():
            pltpu.sync_copy(acc_sp.at[pl.ds(r0, RC), :], obuf)
            pltpu.sync_copy(obuf, out.at[pl.ds(r0, RC), pl.ds(DH * fh, DH)])
        return 0
    lax.fori_loop(0, 8, _write_chunk, 0)

    if compute_deg:
        def _recip_chunk(j, _):
            k = (s * NC + fh) + NS * NC * j

            @pl.when(k < N // RC)
            def _():
                pltpu.sync_copy(deg_sp.at[pl.ds(RC * k, RC)], dbuf)
                for i in range(RC // 16):
                    d = dbuf[pl.ds(16 * i, 16)]
                    rbuf[pl.ds(16 * i, 16)] = 1.0 / jnp.maximum(d, 1.0)
                pltpu.sync_copy(rbuf, recip.at[pl.ds(RC * k, RC)])
            return 0
        lax.fori_loop(0, 4, _recip_chunk, 0)


_agg_scratch_common = dict(
    srcbuf=pltpu.VMEM((KB, EC), jnp.int32),
    dstbuf=pltpu.VMEM((KB, EC), jnp.int32),
    idxbuf=pltpu.VMEM((KB, EC), jnp.int32),
    rows=pltpu.VMEM((KB, EC, DH), jnp.float32),
    zbuf=pltpu.VMEM((125, DH), jnp.float32),
    obuf=pltpu.VMEM((RC, DH), jnp.float32),
)

_agg_deg = pl.kernel(
    functools.partial(_agg_body, True),
    out_type=(jax.ShapeDtypeStruct((N, D), jnp.float32),
              jax.ShapeDtypeStruct((N,), jnp.float32)),
    mesh=_mesh,
    scratch_types=[
        _agg_scratch_common["srcbuf"], _agg_scratch_common["dstbuf"],
        _agg_scratch_common["idxbuf"], _agg_scratch_common["rows"],
        _agg_scratch_common["zbuf"], _agg_scratch_common["obuf"],
        pltpu.VMEM((EC,), jnp.float32),
        pltpu.VMEM((RC,), jnp.float32),
        pltpu.VMEM((RC,), jnp.float32),
        pltpu.VMEM_SHARED((N, DH), jnp.float32),
        pltpu.VMEM_SHARED((N,), jnp.float32),
        pltpu.SemaphoreType.DMA,
        pltpu.SemaphoreType.DMA,
    ],
    compiler_params=_sc_params,
)

_agg_plain = pl.kernel(
    functools.partial(_agg_body, False),
    out_type=jax.ShapeDtypeStruct((N, D), jnp.float32),
    mesh=_mesh,
    scratch_types=[
        _agg_scratch_common["srcbuf"], _agg_scratch_common["dstbuf"],
        _agg_scratch_common["idxbuf"], _agg_scratch_common["rows"],
        _agg_scratch_common["zbuf"], _agg_scratch_common["obuf"],
        pltpu.VMEM_SHARED((N, DH), jnp.float32),
        pltpu.SemaphoreType.DMA,
        pltpu.SemaphoreType.DMA,
    ],
    compiler_params=_sc_params,
)


def _shuffle_body(hxT, perms, shT, hbuf, pbuf, obuf, sem):
    wid = lax.axis_index("s") * NC + lax.axis_index("c")
    for jj in range(4):
        j = wid * 4 + jj
        pltpu.sync_copy(hxT.at[j], hbuf)
        pltpu.sync_copy(perms.at[j], pbuf)

        def _blk(k, _):
            idxv = pbuf[pl.ds(16 * k, 16)]
            obuf[pl.ds(16 * k, 16)] = plsc.load_gather(hbuf, [idxv])
            return 0
        lax.fori_loop(0, N // 16, _blk, 0)
        pltpu.sync_copy(obuf, shT.at[j])


_shuffle = pl.kernel(
    _shuffle_body,
    out_type=jax.ShapeDtypeStruct((D, N), jnp.float32),
    mesh=_mesh,
    scratch_types=[
        pltpu.VMEM((N,), jnp.float32),
        pltpu.VMEM((N,), jnp.int32),
        pltpu.VMEM((N,), jnp.float32),
        pltpu.SemaphoreType.DMA,
    ],
    compiler_params=_sc_params,
)


BN = 1000
GRID = N // BN


def _scale2_body(x1, x2, rec, y1, y2):
    r = rec[...]
    y1[...] = x1[...] * r
    y2[...] = x2[...] * r


_scale2 = pl.pallas_call(
    _scale2_body,
    grid=(GRID,),
    in_specs=[pl.BlockSpec((BN, D), lambda i: (i, 0)),
              pl.BlockSpec((BN, D), lambda i: (i, 0)),
              pl.BlockSpec((BN, 1), lambda i: (i, 0))],
    out_specs=[pl.BlockSpec((BN, D), lambda i: (i, 0)),
               pl.BlockSpec((BN, D), lambda i: (i, 0))],
    out_shape=[jax.ShapeDtypeStruct((N, D), jnp.float32),
               jax.ShapeDtypeStruct((N, D), jnp.float32)],
)


def _gru_body(ax2r, ah2r, rec, hx, Wx, Wh, bb, Wim,
              hxn, Wc, csum):
    r_deg = rec[...]
    ax = ax2r[...] * r_deg
    ah = ah2r[...] * r_deg
    wx = Wx[...]
    wh = Wh[...]
    b = bb[...]
    zr = (jnp.dot(ax, wx[:, :2 * D], preferred_element_type=jnp.float32)
          + jnp.dot(ah, wh[:, :2 * D], preferred_element_type=jnp.float32)
          + b[:, :2 * D])
    z = jax.nn.sigmoid(zr[:, :D])
    r = jax.nn.sigmoid(zr[:, D:])
    ht = jnp.tanh(
        jnp.dot(ax, wx[:, 2 * D:], preferred_element_type=jnp.float32)
        + jnp.dot(r * ah, wh[:, 2 * D:], preferred_element_type=jnp.float32)
        + b[:, 2 * D:])
    hn = (1.0 - z) * hx[...] + z * ht
    hxn[...] = hn
    i = pl.program_id(0)

    @pl.when(i == 0)
    def _():
        csum[...] = jnp.sum(hn, axis=0, keepdims=True)

    @pl.when(i > 0)
    def _():
        csum[...] += jnp.sum(hn, axis=0, keepdims=True)

    @pl.when(i == GRID - 1)
    def _():
        c = jax.nn.sigmoid(csum[...] / N)
        Wc[...] = jnp.dot(c, Wim[...].T, preferred_element_type=jnp.float32)


_gru = pl.pallas_call(
    _gru_body,
    grid=(GRID,),
    in_specs=[pl.BlockSpec((BN, D), lambda i: (i, 0)),
              pl.BlockSpec((BN, D), lambda i: (i, 0)),
              pl.BlockSpec((BN, 1), lambda i: (i, 0)),
              pl.BlockSpec((BN, D), lambda i: (i, 0)),
              pl.BlockSpec((D, 3 * D), lambda i: (0, 0)),
              pl.BlockSpec((D, 3 * D), lambda i: (0, 0)),
              pl.BlockSpec((1, 3 * D), lambda i: (0, 0)),
              pl.BlockSpec((D, D), lambda i: (0, 0))],
    out_specs=[pl.BlockSpec((BN, D), lambda i: (i, 0)),
               pl.BlockSpec((1, D), lambda i: (0, 0))],
    out_shape=[jax.ShapeDtypeStruct((N, D), jnp.float32),
               jax.ShapeDtypeStruct((1, D), jnp.float32)],
    scratch_shapes=[pltpu.VMEM((1, D), jnp.float32)],
)


def _tp_body(x, y):
    y[...] = x[...].T


_tp = pl.pallas_call(
    _tp_body,
    out_shape=jax.ShapeDtypeStruct((D, N), jnp.float32),
)


def _fin_body(hxn, shT, Wc, bim, pos, neg):
    wc = Wc[...]
    bv = bim[0, 0]
    pos[...] = jax.nn.sigmoid(
        jnp.sum(hxn[...] * wc, axis=1, keepdims=True) + bv)
    neg[...] = jax.nn.sigmoid(
        jnp.sum(shT[...] * wc.reshape(D, 1), axis=0, keepdims=True) + bv)


_fin = pl.pallas_call(
    _fin_body,
    out_shape=[jax.ShapeDtypeStruct((N, 1), jnp.float32),
               jax.ShapeDtypeStruct((1, N), jnp.float32)],
)


def _dgi_perms_traced(t):
    base = jax.random.key(42)
    keys = jax.random.split(jax.random.fold_in(base, t), D)
    perms = jax.vmap(lambda k: jax.random.permutation(k, N))(keys)
    return perms.astype(jnp.int32)


@functools.cache
def _dgi_perms_const():
    with jax.ensure_compile_time_eval():
        with jax.default_device(jax.devices("cpu")[0]):
            return [np.asarray(_dgi_perms_traced(t)).astype(np.int32)
                    for t in range(T)]


def _dgi_perms(t):
    try:
        return _dgi_perms_const()[t]
    except Exception:
        return _dgi_perms_traced(t)


def kernel(x_list, edge_index, Wx, Wh, b, W_im, b_im):
    x_list = x_list.astype(jnp.float32)
    Wx = Wx.astype(jnp.float32)
    Wh = Wh.astype(jnp.float32)
    bb = b.astype(jnp.float32).reshape(1, 3 * D)

    hx = jnp.zeros((N, D), jnp.float32)
    zeros_nd = jnp.zeros((N, D), jnp.float32)
    hx_out, pos_out, neg_out = [], [], []
    for t in range(T):
        src = edge_index[t, 0].astype(jnp.int32).reshape(NBLK, EC)
        dst = edge_index[t, 1].astype(jnp.int32).reshape(NBLK, EC)
        x2 = x_list[t].reshape(2 * N, DH)
        s1x, recip = _agg_deg(x2, src, dst)
        if t == 0:
            s1h = zeros_nd
        else:
            s1h = _agg_plain(hx.reshape(2 * N, DH), src, dst)
        rec2d = recip.reshape(N, 1)
        x1n, h1n = _scale2(s1x, s1h, rec2d)
        ax2r = _agg_plain(x1n.reshape(2 * N, DH), src, dst)
        if t == 0:
            ah2r = zeros_nd
        else:
            ah2r = _agg_plain(h1n.reshape(2 * N, DH), src, dst)
        hx, Wc = _gru(ax2r, ah2r, rec2d, hx, Wx, Wh, bb, W_im[t])
        hxT = _tp(hx)

        shT = _shuffle(hxT, _dgi_perms(t))
        pos2d, neg2d = _fin(hx, shT, Wc, b_im[t].reshape(1, 1))
        hx_out.append(hx)
        pos_out.append(pos2d.reshape(N))
        neg_out.append(neg2d.reshape(N))
    return jnp.stack(hx_out), jnp.stack(pos_out), jnp.stack(neg_out)


# speedup vs baseline: 43.4569x; 1.0075x over previous
import functools

import jax
import jax.numpy as jnp
import numpy as np
from jax import lax
from jax.experimental import pallas as pl
from jax.experimental.pallas import tpu as pltpu
from jax.experimental.pallas import tpu_sc as plsc

N = 10000
T = 3
E = 320000
D = 128
DH = 64
NC = 2
NS = 16
EC = 128
NBLK = E // EC
KB = 5
NSS = NBLK // KB
SS_PER_TILE = -(-NSS // NS)
ROWS_PER_TILE = 640
RC = 80

_mesh = plsc.VectorSubcoreMesh(core_axis_name="c", subcore_axis_name="s",
                               num_cores=NC, num_subcores=NS)
_sc_params = pltpu.CompilerParams(use_tc_tiling_on_sc=False,
                                  needs_layout_passes=False)


def _agg_body(compute_deg, h2, src3, dst3, out, *rest):
    if compute_deg:
        (recip, srcbuf, dstbuf, idxbuf, rows, zbuf, obuf, onesbuf, dbuf,
         rbuf, acc_sp, deg_sp, gsem, ssem) = rest
    else:
        (srcbuf, dstbuf, idxbuf, rows, zbuf, obuf, acc_sp, gsem, ssem) = rest
    fh = lax.axis_index("c")
    s = lax.axis_index("s")

    zero16 = jnp.zeros((16,), jnp.float32)
    for k in range(4):
        zbuf[0, pl.ds(16 * k, 16)] = zero16

    def _fill(i, _):
        for k in range(4):
            zbuf[i, pl.ds(16 * k, 16)] = zero16
        return 0
    lax.fori_loop(1, 125, _fill, 0)

    def _zero_chunk(m, _):
        r0 = s * 625 + 125 * m
        pltpu.sync_copy(zbuf, acc_sp.at[pl.ds(r0, 125), :])
        return 0
    lax.fori_loop(0, 5, _zero_chunk, 0)

    if compute_deg:
        for k in range(5):
            rbuf[pl.ds(16 * k, 16)] = zero16

        def _zero_deg(j, _):
            k = s + 16 * j

            @pl.when(k < 125)
            def _():
                pltpu.sync_copy(rbuf, deg_sp.at[pl.ds(RC * k, RC)])
            return 0
        lax.fori_loop(0, 8, _zero_deg, 0)
        ones16 = jnp.ones((16,), jnp.float32)
        for k in range(8):
            onesbuf[pl.ds(16 * k, 16)] = ones16

    plsc.subcore_barrier()

    def _superstep(gi, _):
        g = s + NS * gi

        @pl.when(g < NSS)
        def _():
            pltpu.sync_copy(src3.at[pl.ds(KB * g, KB), :], srcbuf)
            pltpu.sync_copy(dst3.at[pl.ds(KB * g, KB), :], dstbuf)
            for j in range(KB):
                for k in range(EC // 16):
                    idxbuf[j, pl.ds(16 * k, 16)] = (
                        srcbuf[j, pl.ds(16 * k, 16)] * 2 + fh)
            gds = [pltpu.async_copy(h2.at[idxbuf.at[j]], rows.at[j], gsem)
                   for j in range(KB)]
            sds = []
            for j in range(KB):
                gds[j].wait()
                sds.append(pltpu.async_copy(
                    rows.at[j], acc_sp.at[dstbuf.at[j]], ssem, add=True))
                if compute_deg:
                    sds.append(pltpu.async_copy(
                        onesbuf, deg_sp.at[dstbuf.at[j]], ssem, add=True))
            for d in sds:
                d.wait()
        return 0
    lax.fori_loop(0, SS_PER_TILE, _superstep, 0)

    plsc.subcore_barrier()

    def _write_chunk(m, _):
        r0 = s * ROWS_PER_TILE + RC * m

        @pl.when(r0 < N)
        def _():
            pltpu.sync_copy(acc_sp.at[pl.ds(r0, RC), :], obuf)
            pltpu.sync_copy(obuf, out.at[pl.ds(r0, RC), pl.ds(DH * fh, DH)])
        return 0
    lax.fori_loop(0, 8, _write_chunk, 0)

    if compute_deg:
        def _recip_chunk(j, _):
            k = (s * NC + fh) + NS * NC * j

            @pl.when(k < N // RC)
            def _():
                pltpu.sync_copy(deg_sp.at[pl.ds(RC * k, RC)], dbuf)
                for i in range(RC // 16):
                    d = dbuf[pl.ds(16 * i, 16)]
                    rbuf[pl.ds(16 * i, 16)] = 1.0 / jnp.maximum(d, 1.0)
                pltpu.sync_copy(rbuf, recip.at[pl.ds(RC * k, RC)])
            return 0
        lax.fori_loop(0, 4, _recip_chunk, 0)


_agg_scratch_common = dict(
    srcbuf=pltpu.VMEM((KB, EC), jnp.int32),
    dstbuf=pltpu.VMEM((KB, EC), jnp.int32),
    idxbuf=pltpu.VMEM((KB, EC), jnp.int32),
    rows=pltpu.VMEM((KB, EC, DH), jnp.float32),
    zbuf=pltpu.VMEM((125, DH), jnp.float32),
    obuf=pltpu.VMEM((RC, DH), jnp.float32),
)

_agg_deg = pl.kernel(
    functools.partial(_agg_body, True),
    out_type=(jax.ShapeDtypeStruct((N, D), jnp.float32),
              jax.ShapeDtypeStruct((N,), jnp.float32)),
    mesh=_mesh,
    scratch_types=[
        _agg_scratch_common["srcbuf"], _agg_scratch_common["dstbuf"],
        _agg_scratch_common["idxbuf"], _agg_scratch_common["rows"],
        _agg_scratch_common["zbuf"], _agg_scratch_common["obuf"],
        pltpu.VMEM((EC,), jnp.float32),
        pltpu.VMEM((RC,), jnp.float32),
        pltpu.VMEM((RC,), jnp.float32),
        pltpu.VMEM_SHARED((N, DH), jnp.float32),
        pltpu.VMEM_SHARED((N,), jnp.float32),
        pltpu.SemaphoreType.DMA,
        pltpu.SemaphoreType.DMA,
    ],
    compiler_params=_sc_params,
)

_agg_plain = pl.kernel(
    functools.partial(_agg_body, False),
    out_type=jax.ShapeDtypeStruct((N, D), jnp.float32),
    mesh=_mesh,
    scratch_types=[
        _agg_scratch_common["srcbuf"], _agg_scratch_common["dstbuf"],
        _agg_scratch_common["idxbuf"], _agg_scratch_common["rows"],
        _agg_scratch_common["zbuf"], _agg_scratch_common["obuf"],
        pltpu.VMEM_SHARED((N, DH), jnp.float32),
        pltpu.SemaphoreType.DMA,
        pltpu.SemaphoreType.DMA,
    ],
    compiler_params=_sc_params,
)


def _shuffle_body(hxT, perms, shT, hbuf, pbuf, obuf, gsem, wsem):
    wid = lax.axis_index("s") * NC + lax.axis_index("c")
    j0 = wid * 4
    gd = {0: (pltpu.async_copy(hxT.at[j0], hbuf.at[0], gsem),
              pltpu.async_copy(perms.at[j0], pbuf.at[0], gsem))}
    wd = {}
    for jj in range(4):
        p = jj % 2
        if jj < 3:
            q = (jj + 1) % 2
            gd[q] = (pltpu.async_copy(hxT.at[j0 + jj + 1], hbuf.at[q], gsem),
                     pltpu.async_copy(perms.at[j0 + jj + 1], pbuf.at[q],
                                      gsem))
        for d in gd[p]:
            d.wait()
        if p in wd:
            wd[p].wait()

        def _blk(k, _):
            idxv = pbuf[p, pl.ds(16 * k, 16)]
            obuf[p, pl.ds(16 * k, 16)] = plsc.load_gather(hbuf.at[p], [idxv])
            return 0
        lax.fori_loop(0, N // 16, _blk, 0)
        wd[p] = pltpu.async_copy(obuf.at[p], shT.at[j0 + jj], wsem)
    for d in wd.values():
        d.wait()


_shuffle = pl.kernel(
    _shuffle_body,
    out_type=jax.ShapeDtypeStruct((D, N), jnp.float32),
    mesh=_mesh,
    scratch_types=[
        pltpu.VMEM((2, N), jnp.float32),
        pltpu.VMEM((2, N), jnp.int32),
        pltpu.VMEM((2, N), jnp.float32),
        pltpu.SemaphoreType.DMA,
        pltpu.SemaphoreType.DMA,
    ],
    compiler_params=_sc_params,
)


BN = 1000
GRID = N // BN


def _scale2_body(x1, x2, rec, y1, y2):
    r = rec[...]
    y1[...] = x1[...] * r
    y2[...] = x2[...] * r


_scale2 = pl.pallas_call(
    _scale2_body,
    grid=(GRID,),
    in_specs=[pl.BlockSpec((BN, D), lambda i: (i, 0)),
              pl.BlockSpec((BN, D), lambda i: (i, 0)),
              pl.BlockSpec((BN, 1), lambda i: (i, 0))],
    out_specs=[pl.BlockSpec((BN, D), lambda i: (i, 0)),
               pl.BlockSpec((BN, D), lambda i: (i, 0))],
    out_shape=[jax.ShapeDtypeStruct((N, D), jnp.float32),
               jax.ShapeDtypeStruct((N, D), jnp.float32)],
)


def _gru_body(ax2r, ah2r, rec, hx, Wx, Wh, bb, Wim,
              hxn, Wc, csum):
    r_deg = rec[...]
    ax = ax2r[...] * r_deg
    ah = ah2r[...] * r_deg
    wx = Wx[...]
    wh = Wh[...]
    b = bb[...]
    zr = (jnp.dot(ax, wx[:, :2 * D], preferred_element_type=jnp.float32)
          + jnp.dot(ah, wh[:, :2 * D], preferred_element_type=jnp.float32)
          + b[:, :2 * D])
    z = jax.nn.sigmoid(zr[:, :D])
    r = jax.nn.sigmoid(zr[:, D:])
    ht = jnp.tanh(
        jnp.dot(ax, wx[:, 2 * D:], preferred_element_type=jnp.float32)
        + jnp.dot(r * ah, wh[:, 2 * D:], preferred_element_type=jnp.float32)
        + b[:, 2 * D:])
    hn = (1.0 - z) * hx[...] + z * ht
    hxn[...] = hn
    i = pl.program_id(0)

    @pl.when(i == 0)
    def _():
        csum[...] = jnp.sum(hn, axis=0, keepdims=True)

    @pl.when(i > 0)
    def _():
        csum[...] += jnp.sum(hn, axis=0, keepdims=True)

    @pl.when(i == GRID - 1)
    def _():
        c = jax.nn.sigmoid(csum[...] / N)
        Wc[...] = jnp.dot(c, Wim[...].T, preferred_element_type=jnp.float32)


_gru = pl.pallas_call(
    _gru_body,
    grid=(GRID,),
    in_specs=[pl.BlockSpec((BN, D), lambda i: (i, 0)),
              pl.BlockSpec((BN, D), lambda i: (i, 0)),
              pl.BlockSpec((BN, 1), lambda i: (i, 0)),
              pl.BlockSpec((BN, D), lambda i: (i, 0)),
              pl.BlockSpec((D, 3 * D), lambda i: (0, 0)),
              pl.BlockSpec((D, 3 * D), lambda i: (0, 0)),
              pl.BlockSpec((1, 3 * D), lambda i: (0, 0)),
              pl.BlockSpec((D, D), lambda i: (0, 0))],
    out_specs=[pl.BlockSpec((BN, D), lambda i: (i, 0)),
               pl.BlockSpec((1, D), lambda i: (0, 0))],
    out_shape=[jax.ShapeDtypeStruct((N, D), jnp.float32),
               jax.ShapeDtypeStruct((1, D), jnp.float32)],
    scratch_shapes=[pltpu.VMEM((1, D), jnp.float32)],
)


def _tp_body(x, y):
    y[...] = x[...].T


_tp = pl.pallas_call(
    _tp_body,
    out_shape=jax.ShapeDtypeStruct((D, N), jnp.float32),
)


def _fin_body(hxn, shT, Wc, bim, pos, neg):
    wc = Wc[...]
    bv = bim[0, 0]
    pos[...] = jax.nn.sigmoid(
        jnp.sum(hxn[...] * wc, axis=1, keepdims=True) + bv)
    neg[...] = jax.nn.sigmoid(
        jnp.sum(shT[...] * wc.reshape(D, 1), axis=0, keepdims=True) + bv)


_fin = pl.pallas_call(
    _fin_body,
    out_shape=[jax.ShapeDtypeStruct((N, 1), jnp.float32),
               jax.ShapeDtypeStruct((1, N), jnp.float32)],
)


def _dgi_perms_traced(t):
    base = jax.random.key(42)
    keys = jax.random.split(jax.random.fold_in(base, t), D)
    perms = jax.vmap(lambda k: jax.random.permutation(k, N))(keys)
    return perms.astype(jnp.int32)


@functools.cache
def _dgi_perms_const():
    with jax.ensure_compile_time_eval():
        with jax.default_device(jax.devices("cpu")[0]):
            return [np.asarray(_dgi_perms_traced(t)).astype(np.int32)
                    for t in range(T)]


def _dgi_perms(t):
    try:
        return _dgi_perms_const()[t]
    except Exception:
        return _dgi_perms_traced(t)


def kernel(x_list, edge_index, Wx, Wh, b, W_im, b_im):
    x_list = x_list.astype(jnp.float32)
    Wx = Wx.astype(jnp.float32)
    Wh = Wh.astype(jnp.float32)
    bb = b.astype(jnp.float32).reshape(1, 3 * D)

    hx = jnp.zeros((N, D), jnp.float32)
    zeros_nd = jnp.zeros((N, D), jnp.float32)
    hx_out, pos_out, neg_out = [], [], []
    for t in range(T):
        src = edge_index[t, 0].astype(jnp.int32).reshape(NBLK, EC)
        dst = edge_index[t, 1].astype(jnp.int32).reshape(NBLK, EC)
        x2 = x_list[t].reshape(2 * N, DH)
        s1x, recip = _agg_deg(x2, src, dst)
        if t == 0:
            s1h = zeros_nd
        else:
            s1h = _agg_plain(hx.reshape(2 * N, DH), src, dst)
        rec2d = recip.reshape(N, 1)
        x1n, h1n = _scale2(s1x, s1h, rec2d)
        ax2r = _agg_plain(x1n.reshape(2 * N, DH), src, dst)
        if t == 0:
            ah2r = zeros_nd
        else:
            ah2r = _agg_plain(h1n.reshape(2 * N, DH), src, dst)
        hx, Wc = _gru(ax2r, ah2r, rec2d, hx, Wx, Wh, bb, W_im[t])
        hxT = _tp(hx)

        shT = _shuffle(hxT, _dgi_perms(t))
        pos2d, neg2d = _fin(hx, shT, Wc, b_im[t].reshape(1, 1))
        hx_out.append(hx)
        pos_out.append(pos2d.reshape(N))
        neg_out.append(neg2d.reshape(N))
    return jnp.stack(hx_out), jnp.stack(pos_out), jnp.stack(neg_out)
